# baseline (speedup 1.0000x reference)
import functools

import jax
import jax.numpy as jnp
from jax import lax
from jax.experimental import pallas as pl
from jax.experimental.pallas import tpu as pltpu
from jax.experimental.pallas import tpu_sc as plsc

B, L = 4096, 50
N = B * L
D_W, D_F, D_OUT = 32, 16, 64
NC, NS = 2, 16
NW = NC * NS
ROWS_PER_W = N // NW
CHUNK = 128
NCHUNK = ROWS_PER_W // CHUNK
NGRP = CHUNK // 16

V = 1000000
TILES = 7813
TPW = TILES // NW
XTRA = TILES - TPW * NW
WROWS_PAD = TILES * 32

_mesh = plsc.VectorSubcoreMesh(
    core_axis_name="c", subcore_axis_name="s", num_cores=NC, num_subcores=NS
)
_params = pltpu.CompilerParams(use_tc_tiling_on_sc=True,
                               needs_layout_passes=False)


@functools.partial(
    pl.kernel,
    out_type=jax.ShapeDtypeStruct((WROWS_PAD, 128), jnp.float32),
    mesh=_mesh,
    compiler_params=_params,
    scratch_types=[
        [pltpu.VMEM((32, 128), jnp.float32) for _ in range(2)],
        [pltpu.VMEM((32, 128), jnp.float32) for _ in range(2)],
        [pltpu.SemaphoreType.DMA for _ in range(2)],
        [pltpu.SemaphoreType.DMA for _ in range(2)],
    ],
)
def _cvt_kernel(wvt, out, inb, ob, rsems, wsems):
    wid = lax.axis_index("s") * NC + lax.axis_index("c")
    nblk = TPW + jnp.where(wid < XTRA, 1, 0)
    start = wid * TPW + jnp.minimum(wid, XTRA)
    end = start + nblk
    iota = lax.iota(jnp.int32, 16)

    def read(tile, p):
        pltpu.async_copy(
            wvt.at[pl.ds(0, 32), pl.ds(tile * 128, 128)], inb[p], rsems[p])

    def drain_read(p):
        pltpu.make_async_copy(
            wvt.at[pl.ds(0, 32), pl.ds(0, 128)], inb[p], rsems[p]).wait()

    def drain_write(p):
        pltpu.make_async_copy(
            ob[p], out.at[pl.ds(0, 32)], wsems[p]).wait()

    def transpose(p):
        def grp(g, carry):
            for k in range(16):
                wl = g * 16 + k
                wlv = jnp.full((16,), wl, jnp.int32)
                lo = plsc.load_gather(inb[p], [iota, wlv])
                hi = plsc.load_gather(inb[p], [iota + 16, wlv])
                r = 4 * g + (k // 4)
                c = (k % 4) * 32
                ob[p][r, pl.ds(c, 16)] = lo
                ob[p][r, pl.ds(c + 16, 16)] = hi
            return carry
        lax.fori_loop(0, 8, grp, 0)

    read(start, 0)
    read(start + 1, 1)

    def pair_body(t, carry):
        t0 = start + 2 * t
        for p in (0, 1):
            tt = t0 + p

            @pl.when(tt < end)
            def _():
                drain_read(p)

                @pl.when(tt - 2 >= start)
                def _():
                    drain_write(p)

                transpose(p)
                pltpu.async_copy(ob[p], out.at[pl.ds(tt * 32, 32)], wsems[p])

                @pl.when(tt + 2 < end)
                def _():
                    read(tt + 2, p)
        return carry

    npair = (TPW + 2) // 2
    lax.fori_loop(0, npair, pair_body, 0)
    for p in (0, 1):
        drain_write(p)


@functools.partial(
    pl.kernel,
    out_type=jax.ShapeDtypeStruct((N * D_OUT,), jnp.float32),
    mesh=_mesh,
    compiler_params=_params,
    scratch_types=[
        pltpu.VMEM((ROWS_PER_W,), jnp.int32),
        pltpu.VMEM((ROWS_PER_W,), jnp.int32),
        pltpu.VMEM((ROWS_PER_W,), jnp.int32),
        pltpu.VMEM((ROWS_PER_W,), jnp.int32),
        pltpu.VMEM((16000,), jnp.float32),
        pltpu.VMEM((16000,), jnp.float32),
        pltpu.VMEM((CHUNK, 128), jnp.float32),
        pltpu.VMEM((CHUNK * D_OUT,), jnp.float32),
        pltpu.SemaphoreType.DMA,
    ],
)
def _emb_kernel(xq, xo, lo, ro, wv, pf1, pf2, out, qv, ov, lv, rv,
                pf1v, pf2v, wbuf, obuf, sem):
    wid = lax.axis_index("s") * NC + lax.axis_index("c")
    base = wid * ROWS_PER_W
    rows = pl.ds(base, ROWS_PER_W)
    pltpu.sync_copy(xq.at[rows], qv)
    pltpu.sync_copy(xo.at[rows], ov)
    pltpu.sync_copy(lo.at[rows], lv)
    pltpu.sync_copy(ro.at[rows], rv)
    pltpu.sync_copy(pf1, pf1v)
    pltpu.sync_copy(pf2, pf2v)
    iota = lax.iota(jnp.int32, 16)

    def chunk_body(ci, carry):
        c0 = ci * CHUNK
        pltpu.async_copy(wv.at[qv.at[pl.ds(c0, CHUNK)]], wbuf, sem).wait()

        def grp_body(g, carry2):
            i0 = g * 16
            offv = ov[pl.ds(c0 + i0, 16)]
            lofv = lv[pl.ds(c0 + i0, 16)]
            rofv = rv[pl.ds(c0 + i0, 16)]
            for k in range(16):
                i = i0 + k
                ri = jnp.full((16,), i, jnp.int32)
                cw = offv[k] + iota
                g0 = plsc.load_gather(wbuf, [ri, cw])
                g1 = plsc.load_gather(wbuf, [ri, cw + 16])
                gl = plsc.load_gather(pf1v, [lofv[k] + iota])
                gr = plsc.load_gather(pf2v, [rofv[k] + iota])
                obuf[pl.ds(i * D_OUT, 16)] = g0
                obuf[pl.ds(i * D_OUT + 16, 16)] = g1
                obuf[pl.ds(i * D_OUT + 32, 16)] = gl
                obuf[pl.ds(i * D_OUT + 48, 16)] = gr
            return carry2

        lax.fori_loop(0, NGRP, grp_body, 0)
        pltpu.sync_copy(obuf, out.at[pl.ds((base + c0) * D_OUT, CHUNK * D_OUT)])
        return carry

    lax.fori_loop(0, NCHUNK, chunk_body, 0)


def kernel(x, ldist, rdist, Wv, pf1, pf2):
    xi = x.reshape(-1).astype(jnp.int32)
    li = ldist.reshape(-1).astype(jnp.int32)
    ri = rdist.reshape(-1).astype(jnp.int32)
    xq = xi >> 2
    xo = (xi & 3) * D_W
    lo = li * D_F
    ro = ri * D_F
    wv128 = _cvt_kernel(Wv.T)
    out = _emb_kernel(xq, xo, lo, ro, wv128,
                      pf1.reshape(-1), pf2.reshape(-1))
    return out.reshape(B, 1, L, D_OUT)


# speedup vs baseline: 1.1931x; 1.1931x over previous
import functools

import jax
import jax.numpy as jnp
from jax import lax
from jax.experimental import pallas as pl
from jax.experimental.pallas import tpu as pltpu
from jax.experimental.pallas import tpu_sc as plsc

B, L = 4096, 50
N = B * L
D_W, D_F, D_OUT = 32, 16, 64
NC, NS = 2, 16
NW = NC * NS
ROWS_PER_W = N // NW
CHUNK = 128
NCHUNK = ROWS_PER_W // CHUNK
NGRP = CHUNK // 16

V = 1000000
TILES = 7813
TPW = TILES // NW
XTRA = TILES - TPW * NW
WROWS_PAD = TILES * 32

_mesh = plsc.VectorSubcoreMesh(
    core_axis_name="c", subcore_axis_name="s", num_cores=NC, num_subcores=NS
)
_params = pltpu.CompilerParams(use_tc_tiling_on_sc=True,
                               needs_layout_passes=False)


@functools.partial(
    pl.kernel,
    out_type=jax.ShapeDtypeStruct((WROWS_PAD, 128), jnp.float32),
    mesh=_mesh,
    compiler_params=_params,
    scratch_types=[
        [pltpu.VMEM((32, 128), jnp.float32) for _ in range(2)],
        [pltpu.VMEM((32, 128), jnp.float32) for _ in range(2)],
        [pltpu.SemaphoreType.DMA for _ in range(2)],
        [pltpu.SemaphoreType.DMA for _ in range(2)],
    ],
)
def _cvt_kernel(wvt, out, inb, ob, rsems, wsems):
    wid = lax.axis_index("s") * NC + lax.axis_index("c")
    nblk = TPW + jnp.where(wid < XTRA, 1, 0)
    start = wid * TPW + jnp.minimum(wid, XTRA)
    end = start + nblk
    iota = lax.iota(jnp.int32, 16)

    def read(tile, p):
        pltpu.async_copy(
            wvt.at[pl.ds(0, 32), pl.ds(tile * 128, 128)], inb[p], rsems[p])

    def drain_read(p):
        pltpu.make_async_copy(
            wvt.at[pl.ds(0, 32), pl.ds(0, 128)], inb[p], rsems[p]).wait()

    def drain_write(p):
        pltpu.make_async_copy(
            ob[p], out.at[pl.ds(0, 32)], wsems[p]).wait()

    def transpose(p):
        def grp(g, carry):
            vals = []
            for k in range(16):
                wl = g * 16 + k
                wlv = jnp.full((16,), wl, jnp.int32)
                vals.append((plsc.load_gather(inb[p], [iota, wlv]),
                             plsc.load_gather(inb[p], [iota + 16, wlv])))
            for k in range(16):
                r = 4 * g + (k // 4)
                c = (k % 4) * 32
                ob[p][r, pl.ds(c, 16)] = vals[k][0]
                ob[p][r, pl.ds(c + 16, 16)] = vals[k][1]
            return carry
        lax.fori_loop(0, 8, grp, 0)

    read(start, 0)
    read(start + 1, 1)

    def pair_body(t, carry):
        t0 = start + 2 * t
        for p in (0, 1):
            tt = t0 + p

            @pl.when(tt < end)
            def _():
                drain_read(p)

                @pl.when(tt - 2 >= start)
                def _():
                    drain_write(p)

                transpose(p)
                pltpu.async_copy(ob[p], out.at[pl.ds(tt * 32, 32)], wsems[p])

                @pl.when(tt + 2 < end)
                def _():
                    read(tt + 2, p)
        return carry

    npair = (TPW + 2) // 2
    lax.fori_loop(0, npair, pair_body, 0)
    for p in (0, 1):
        drain_write(p)


@functools.partial(
    pl.kernel,
    out_type=jax.ShapeDtypeStruct((N * D_OUT,), jnp.float32),
    mesh=_mesh,
    compiler_params=_params,
    scratch_types=[
        pltpu.VMEM((ROWS_PER_W,), jnp.int32),
        pltpu.VMEM((ROWS_PER_W,), jnp.int32),
        pltpu.VMEM((ROWS_PER_W,), jnp.int32),
        pltpu.VMEM((ROWS_PER_W,), jnp.int32),
        pltpu.VMEM((16000,), jnp.float32),
        pltpu.VMEM((16000,), jnp.float32),
        pltpu.VMEM((CHUNK, 128), jnp.float32),
        pltpu.VMEM((CHUNK * D_OUT,), jnp.float32),
        pltpu.SemaphoreType.DMA,
    ],
)
def _emb_kernel(xq, xo, lo, ro, wv, pf1, pf2, out, qv, ov, lv, rv,
                pf1v, pf2v, wbuf, obuf, sem):
    wid = lax.axis_index("s") * NC + lax.axis_index("c")
    base = wid * ROWS_PER_W
    rows = pl.ds(base, ROWS_PER_W)
    pltpu.sync_copy(xq.at[rows], qv)
    pltpu.sync_copy(xo.at[rows], ov)
    pltpu.sync_copy(lo.at[rows], lv)
    pltpu.sync_copy(ro.at[rows], rv)
    pltpu.sync_copy(pf1, pf1v)
    pltpu.sync_copy(pf2, pf2v)
    iota = lax.iota(jnp.int32, 16)

    def chunk_body(ci, carry):
        c0 = ci * CHUNK
        pltpu.async_copy(wv.at[qv.at[pl.ds(c0, CHUNK)]], wbuf, sem).wait()

        def grp_body(g, carry2):
            i0 = g * 16
            offv = ov[pl.ds(c0 + i0, 16)]
            lofv = lv[pl.ds(c0 + i0, 16)]
            rofv = rv[pl.ds(c0 + i0, 16)]
            for k in range(16):
                i = i0 + k
                ri = jnp.full((16,), i, jnp.int32)
                cw = offv[k] + iota
                g0 = plsc.load_gather(wbuf, [ri, cw])
                g1 = plsc.load_gather(wbuf, [ri, cw + 16])
                gl = plsc.load_gather(pf1v, [lofv[k] + iota])
                gr = plsc.load_gather(pf2v, [rofv[k] + iota])
                obuf[pl.ds(i * D_OUT, 16)] = g0
                obuf[pl.ds(i * D_OUT + 16, 16)] = g1
                obuf[pl.ds(i * D_OUT + 32, 16)] = gl
                obuf[pl.ds(i * D_OUT + 48, 16)] = gr
            return carry2

        lax.fori_loop(0, NGRP, grp_body, 0)
        pltpu.sync_copy(obuf, out.at[pl.ds((base + c0) * D_OUT, CHUNK * D_OUT)])
        return carry

    lax.fori_loop(0, NCHUNK, chunk_body, 0)


def kernel(x, ldist, rdist, Wv, pf1, pf2):
    xi = x.reshape(-1).astype(jnp.int32)
    li = ldist.reshape(-1).astype(jnp.int32)
    ri = rdist.reshape(-1).astype(jnp.int32)
    xq = xi >> 2
    xo = (xi & 3) * D_W
    lo = li * D_F
    ro = ri * D_F
    wv128 = _cvt_kernel(Wv.T)
    out = _emb_kernel(xq, xo, lo, ro, wv128,
                      pf1.reshape(-1), pf2.reshape(-1))
    return out.reshape(B, 1, L, D_OUT)


# speedup vs baseline: 1.1967x; 1.0030x over previous
import functools

import jax
import jax.numpy as jnp
from jax import lax
from jax.experimental import pallas as pl
from jax.experimental.pallas import tpu as pltpu
from jax.experimental.pallas import tpu_sc as plsc

B, L = 4096, 50
N = B * L
D_W, D_F, D_OUT = 32, 16, 64
NC, NS = 2, 16
NW = NC * NS
ROWS_PER_W = N // NW
CHUNK = 128
NCHUNK = ROWS_PER_W // CHUNK
NGRP = CHUNK // 16

V = 1000000
TILES = 7813
TPW = TILES // NW
XTRA = TILES - TPW * NW
WROWS_PAD = TILES * 32

_mesh = plsc.VectorSubcoreMesh(
    core_axis_name="c", subcore_axis_name="s", num_cores=NC, num_subcores=NS
)
_params = pltpu.CompilerParams(use_tc_tiling_on_sc=True,
                               needs_layout_passes=False)


@functools.partial(
    pl.kernel,
    out_type=jax.ShapeDtypeStruct((WROWS_PAD, 128), jnp.float32),
    mesh=_mesh,
    compiler_params=_params,
    scratch_types=[
        [pltpu.VMEM((32, 128), jnp.float32) for _ in range(2)],
        [pltpu.VMEM((32, 128), jnp.float32) for _ in range(2)],
        [pltpu.SemaphoreType.DMA for _ in range(2)],
        [pltpu.SemaphoreType.DMA for _ in range(2)],
    ],
)
def _cvt_kernel(wvt, out, inb, ob, rsems, wsems):
    wid = lax.axis_index("s") * NC + lax.axis_index("c")
    nblk = TPW + jnp.where(wid < XTRA, 1, 0)
    start = wid * TPW + jnp.minimum(wid, XTRA)
    end = start + nblk
    iota = lax.iota(jnp.int32, 16)

    def read(tile, p):
        pltpu.async_copy(
            wvt.at[pl.ds(0, 32), pl.ds(tile * 128, 128)], inb[p], rsems[p])

    def drain_read(p):
        pltpu.make_async_copy(
            wvt.at[pl.ds(0, 32), pl.ds(0, 128)], inb[p], rsems[p]).wait()

    def drain_write(p):
        pltpu.make_async_copy(
            ob[p], out.at[pl.ds(0, 32)], wsems[p]).wait()

    def transpose(p):
        for g in range(8):
            vals = []
            for k in range(16):
                wl = g * 16 + k
                wlv = jnp.full((16,), wl, jnp.int32)
                vals.append((plsc.load_gather(inb[p], [iota, wlv]),
                             plsc.load_gather(inb[p], [iota + 16, wlv])))
            for k in range(16):
                r = 4 * g + (k // 4)
                c = (k % 4) * 32
                ob[p][r, pl.ds(c, 16)] = vals[k][0]
                ob[p][r, pl.ds(c + 16, 16)] = vals[k][1]

    read(start, 0)
    read(start + 1, 1)

    def pair_body(t, carry):
        t0 = start + 2 * t
        for p in (0, 1):
            tt = t0 + p

            @pl.when(tt < end)
            def _():
                drain_read(p)

                @pl.when(tt - 2 >= start)
                def _():
                    drain_write(p)

                transpose(p)
                pltpu.async_copy(ob[p], out.at[pl.ds(tt * 32, 32)], wsems[p])

                @pl.when(tt + 2 < end)
                def _():
                    read(tt + 2, p)
        return carry

    npair = (TPW + 2) // 2
    lax.fori_loop(0, npair, pair_body, 0)
    for p in (0, 1):
        drain_write(p)


@functools.partial(
    pl.kernel,
    out_type=jax.ShapeDtypeStruct((N * D_OUT,), jnp.float32),
    mesh=_mesh,
    compiler_params=_params,
    scratch_types=[
        pltpu.VMEM((ROWS_PER_W,), jnp.int32),
        pltpu.VMEM((ROWS_PER_W,), jnp.int32),
        pltpu.VMEM((ROWS_PER_W,), jnp.int32),
        pltpu.VMEM((ROWS_PER_W,), jnp.int32),
        pltpu.VMEM((16000,), jnp.float32),
        pltpu.VMEM((16000,), jnp.float32),
        pltpu.VMEM((CHUNK, 128), jnp.float32),
        pltpu.VMEM((CHUNK * D_OUT,), jnp.float32),
        pltpu.SemaphoreType.DMA,
    ],
)
def _emb_kernel(xq, xo, lo, ro, wv, pf1, pf2, out, qv, ov, lv, rv,
                pf1v, pf2v, wbuf, obuf, sem):
    wid = lax.axis_index("s") * NC + lax.axis_index("c")
    base = wid * ROWS_PER_W
    rows = pl.ds(base, ROWS_PER_W)
    pltpu.sync_copy(xq.at[rows], qv)
    pltpu.sync_copy(xo.at[rows], ov)
    pltpu.sync_copy(lo.at[rows], lv)
    pltpu.sync_copy(ro.at[rows], rv)
    pltpu.sync_copy(pf1, pf1v)
    pltpu.sync_copy(pf2, pf2v)
    iota = lax.iota(jnp.int32, 16)

    def chunk_body(ci, carry):
        c0 = ci * CHUNK
        pltpu.async_copy(wv.at[qv.at[pl.ds(c0, CHUNK)]], wbuf, sem).wait()

        def grp_body(g, carry2):
            i0 = g * 16
            offv = ov[pl.ds(c0 + i0, 16)]
            lofv = lv[pl.ds(c0 + i0, 16)]
            rofv = rv[pl.ds(c0 + i0, 16)]
            for k in range(16):
                i = i0 + k
                ri = jnp.full((16,), i, jnp.int32)
                cw = offv[k] + iota
                g0 = plsc.load_gather(wbuf, [ri, cw])
                g1 = plsc.load_gather(wbuf, [ri, cw + 16])
                gl = plsc.load_gather(pf1v, [lofv[k] + iota])
                gr = plsc.load_gather(pf2v, [rofv[k] + iota])
                obuf[pl.ds(i * D_OUT, 16)] = g0
                obuf[pl.ds(i * D_OUT + 16, 16)] = g1
                obuf[pl.ds(i * D_OUT + 32, 16)] = gl
                obuf[pl.ds(i * D_OUT + 48, 16)] = gr
            return carry2

        lax.fori_loop(0, NGRP, grp_body, 0)
        pltpu.sync_copy(obuf, out.at[pl.ds((base + c0) * D_OUT, CHUNK * D_OUT)])
        return carry

    lax.fori_loop(0, NCHUNK, chunk_body, 0)


def kernel(x, ldist, rdist, Wv, pf1, pf2):
    xi = x.reshape(-1).astype(jnp.int32)
    li = ldist.reshape(-1).astype(jnp.int32)
    ri = rdist.reshape(-1).astype(jnp.int32)
    xq = xi >> 2
    xo = (xi & 3) * D_W
    lo = li * D_F
    ro = ri * D_F
    wv128 = _cvt_kernel(Wv.T)
    out = _emb_kernel(xq, xo, lo, ro, wv128,
                      pf1.reshape(-1), pf2.reshape(-1))
    return out.reshape(B, 1, L, D_OUT)


# speedup vs baseline: 1.3580x; 1.1348x over previous
import functools

import jax
import jax.numpy as jnp
from jax import lax
from jax.experimental import pallas as pl
from jax.experimental.pallas import tpu as pltpu
from jax.experimental.pallas import tpu_sc as plsc

B, L = 4096, 50
N = B * L
D_W, D_F, D_OUT = 32, 16, 64
NC, NS = 2, 16
NW = NC * NS
ROWS_PER_W = N // NW
CHUNK = 640
NCHUNK = ROWS_PER_W // CHUNK
NBUF = 2

_mesh = plsc.VectorSubcoreMesh(
    core_axis_name="c", subcore_axis_name="s", num_cores=NC, num_subcores=NS
)


@functools.partial(
    pl.kernel,
    out_type=jax.ShapeDtypeStruct((N, D_OUT), jnp.float32),
    mesh=_mesh,
    compiler_params=pltpu.CompilerParams(use_tc_tiling_on_sc=False),
    scratch_types=[
        pltpu.VMEM((ROWS_PER_W,), jnp.int32),
        pltpu.VMEM((ROWS_PER_W,), jnp.int32),
        pltpu.VMEM((ROWS_PER_W,), jnp.int32),
        [pltpu.VMEM((CHUNK, D_W), jnp.float32) for _ in range(NBUF)],
        [pltpu.VMEM((CHUNK, D_F), jnp.float32) for _ in range(NBUF)],
        [pltpu.VMEM((CHUNK, D_F), jnp.float32) for _ in range(NBUF)],
        [pltpu.SemaphoreType.DMA for _ in range(NBUF)],
        [pltpu.SemaphoreType.DMA for _ in range(NBUF)],
    ],
)
def _emb_kernel(xi, li, ri, wv, pf1, pf2, out, xidx, lidx, ridx,
                wbufs, lbufs, rbufs, gsems, wsems):
    wid = lax.axis_index("s") * NC + lax.axis_index("c")
    base = wid * ROWS_PER_W
    all_rows = pl.ds(base, ROWS_PER_W)
    pltpu.sync_copy(xi.at[all_rows], xidx)
    pltpu.sync_copy(li.at[all_rows], lidx)
    pltpu.sync_copy(ri.at[all_rows], ridx)

    def issue_gathers(ci, b):
        idx = pl.ds(ci * CHUNK, CHUNK)
        return [
            pltpu.async_copy(wv.at[xidx.at[idx]], wbufs[b], gsems[b]),
            pltpu.async_copy(pf1.at[lidx.at[idx]], lbufs[b], gsems[b]),
            pltpu.async_copy(pf2.at[ridx.at[idx]], rbufs[b], gsems[b]),
        ]

    def issue_writes(ci, b):
        rows = pl.ds(base + ci * CHUNK, CHUNK)
        return [
            pltpu.async_copy(wbufs[b], out.at[rows, pl.ds(0, D_W)], wsems[b]),
            pltpu.async_copy(lbufs[b], out.at[rows, pl.ds(D_W, D_F)], wsems[b]),
            pltpu.async_copy(rbufs[b], out.at[rows, pl.ds(D_W + D_F, D_F)], wsems[b]),
        ]

    gathers = {0: issue_gathers(0, 0)}
    writes = {}
    for ci in range(NCHUNK):
        b = ci % NBUF
        if ci + 1 < NCHUNK:
            if ci >= 1:
                for cp in writes[ci - 1]:
                    cp.wait()
            gathers[ci + 1] = issue_gathers(ci + 1, (ci + 1) % NBUF)
        for cp in gathers[ci]:
            cp.wait()
        writes[ci] = issue_writes(ci, b)
    for cp in writes[NCHUNK - 1]:
        cp.wait()
    for cp in writes[NCHUNK - 2]:
        cp.wait()


def kernel(x, ldist, rdist, Wv, pf1, pf2):
    xi = x.reshape(-1).astype(jnp.int32)
    li = ldist.reshape(-1).astype(jnp.int32)
    ri = rdist.reshape(-1).astype(jnp.int32)
    out = _emb_kernel(xi, li, ri, Wv, pf1, pf2)
    return out.reshape(B, 1, L, D_OUT)
